# baseline (speedup 1.0000x reference)
import functools

import jax
import jax.numpy as jnp
from jax import lax
from jax.experimental import pallas as pl
from jax.experimental.pallas import tpu as pltpu
from jax.experimental.pallas import tpu_sc as plsc

CH = 128
R = 8
G = 4


@functools.lru_cache(maxsize=None)
def _make_detile(hist_padded, batch):
    ht = hist_padded // 8

    def body(x_ref, o_ref):
        o_ref[...] = x_ref[...].reshape(8 * batch)

    return pl.pallas_call(
        body,
        grid=(ht,),
        in_specs=[pl.BlockSpec((8, batch), lambda i: (i, 0))],
        out_specs=pl.BlockSpec((8 * batch,), lambda i: (i,)),
        out_shape=jax.ShapeDtypeStruct((hist_padded * batch,), jnp.int32),
    )


@functools.lru_cache(maxsize=None)
def _make_gather(hist, hist_padded, batch, dim):
    mesh = plsc.VectorSubcoreMesh(core_axis_name="c", subcore_axis_name="s")
    nc, ns = mesh.num_cores, mesh.num_subcores
    nw = nc * ns
    bt = batch // CH
    num_chunks = hist * bt
    assert num_chunks % nw == 0
    chunks_per_w = num_chunks // nw
    steady = chunks_per_w - 2 * G
    assert steady % R == 0 and chunks_per_w > 2 * R

    @functools.partial(
        pl.kernel,
        out_type=jax.ShapeDtypeStruct((hist, batch, dim), jnp.float32),
        mesh=mesh,
        scratch_types=[
            pltpu.VMEM((chunks_per_w * CH,), jnp.int32),
            pltpu.VMEM((R, CH, dim), jnp.float32),
            pltpu.SemaphoreType.DMA((R,)),
            pltpu.SemaphoreType.DMA((R,)),
        ],
        compiler_params=pltpu.CompilerParams(use_tc_tiling_on_sc=False),
    )
    def gather_kernel(idx_hbm, table_hbm, out_hbm, idx_v, rows_v, gsem, ssem):
        wid = lax.axis_index("s") * nc + lax.axis_index("c")
        row0 = wid * chunks_per_w
        pltpu.sync_copy(
            idx_hbm.at[pl.ds(row0 * CH, chunks_per_w * CH)], idx_v
        )

        def fire_gather(t, rr):
            pltpu.async_copy(
                table_hbm.at[idx_v.at[pl.ds(t * CH, CH)]],
                rows_v.at[rr],
                gsem.at[rr],
            )

        def wait_gather(rr):
            pltpu.make_async_copy(
                out_hbm.at[0, pl.ds(0, CH)], rows_v.at[rr], gsem.at[rr]
            ).wait()

        def fire_scatter(t, rr):
            r = row0 + t
            h = r // bt
            tb = r % bt
            pltpu.async_copy(
                rows_v.at[rr],
                out_hbm.at[h, pl.ds(tb * CH, CH)],
                ssem.at[rr],
            )

        def wait_scatter(rr):
            pltpu.make_async_copy(
                rows_v.at[rr], out_hbm.at[0, pl.ds(0, CH)], ssem.at[rr]
            ).wait()

        for t in range(R - G):
            fire_gather(t, t % R)
        for i in range(G):
            fire_gather(i + (R - G), (i + (R - G)) % R)
            wait_gather(i % R)
            fire_scatter(i, i % R)

        @pl.loop(0, steady // R)
        def _(o):
            t0 = G + o * R
            for k in range(R):
                t = t0 + k
                bpre = (G + k + (R - G)) % R
                wait_scatter(bpre)
                fire_gather(t + (R - G), bpre)
                b = (G + k) % R
                wait_gather(b)
                fire_scatter(t, b)

        for t in range(chunks_per_w - G, chunks_per_w):
            b = t % R
            wait_gather(b)
            fire_scatter(t, b)
        for rr in range(R):
            wait_scatter(rr)

    return gather_kernel


def kernel(x, weight):
    b, h = x.shape
    n, d = weight.shape
    hp = ((h + 7) // 8) * 8
    xp = jnp.pad(x, ((0, 0), (0, hp - h))) if hp != h else x
    xt = jnp.swapaxes(xp, 0, 1).astype(jnp.int32)
    idx = _make_detile(hp, b)(xt)
    out = _make_gather(h, hp, b, d)(idx, weight)
    return out.transpose(1, 0, 2)


# speedup vs baseline: 1.0825x; 1.0825x over previous
import functools

import jax
import jax.numpy as jnp
from jax import lax
from jax.experimental import pallas as pl
from jax.experimental.pallas import tpu as pltpu
from jax.experimental.pallas import tpu_sc as plsc

CH = 128
R = 8
G = 4


BLK = 2048


@functools.lru_cache(maxsize=None)
def _make_detile(hist_padded, batch, dim):
    ht = hist_padded // 8
    g = 128 // dim
    s = BLK // g
    sh = s.bit_length() - 1

    def body(x_ref, o_ref):
        r = x_ref[...]
        rp = (
            (r & ~(BLK - 1))
            + ((r & (s - 1)) << (g.bit_length() - 1))
            + ((r & (BLK - 1)) >> sh)
        )
        o_ref[...] = rp.reshape(8 * batch)

    return pl.pallas_call(
        body,
        grid=(ht,),
        in_specs=[pl.BlockSpec((8, batch), lambda i: (i, 0))],
        out_specs=pl.BlockSpec((8 * batch,), lambda i: (i,)),
        out_shape=jax.ShapeDtypeStruct((hist_padded * batch,), jnp.int32),
    )


@functools.lru_cache(maxsize=None)
def _make_detranspose(nemb, dim):
    g128 = 128 // dim
    s = BLK // g128
    grid = (nemb + BLK - 1) // BLK

    def body(w_ref, o_ref):
        t = jnp.swapaxes(w_ref[...], 0, 1)
        merged = jnp.concatenate(
            [t[k * s:(k + 1) * s] for k in range(g128)], axis=1
        )
        o_ref[...] = merged.reshape(BLK * dim)

    return pl.pallas_call(
        body,
        grid=(grid,),
        in_specs=[pl.BlockSpec((dim, BLK), lambda i: (0, i))],
        out_specs=pl.BlockSpec((BLK * dim,), lambda i: (i,)),
        out_shape=jax.ShapeDtypeStruct((grid * BLK * dim,), jnp.float32),
    )


@functools.lru_cache(maxsize=None)
def _make_gather(hist, hist_padded, batch, dim):
    mesh = plsc.VectorSubcoreMesh(core_axis_name="c", subcore_axis_name="s")
    nc, ns = mesh.num_cores, mesh.num_subcores
    nw = nc * ns
    bt = batch // CH
    num_chunks = hist * bt
    assert num_chunks % nw == 0
    chunks_per_w = num_chunks // nw
    steady = chunks_per_w - 2 * G
    assert steady % R == 0 and chunks_per_w > 2 * R

    @functools.partial(
        pl.kernel,
        out_type=jax.ShapeDtypeStruct((hist, batch, dim), jnp.float32),
        mesh=mesh,
        scratch_types=[
            pltpu.VMEM((chunks_per_w * CH,), jnp.int32),
            pltpu.VMEM((R, CH, dim), jnp.float32),
            pltpu.SemaphoreType.DMA((R,)),
            pltpu.SemaphoreType.DMA((R,)),
        ],
        compiler_params=pltpu.CompilerParams(use_tc_tiling_on_sc=False),
    )
    def gather_kernel(idx_hbm, table_hbm, out_hbm, idx_v, rows_v, gsem, ssem):
        wid = lax.axis_index("s") * nc + lax.axis_index("c")
        row0 = wid * chunks_per_w
        pltpu.sync_copy(
            idx_hbm.at[pl.ds(row0 * CH, chunks_per_w * CH)], idx_v
        )

        def fire_gather(t, rr):
            pltpu.async_copy(
                table_hbm.at[idx_v.at[pl.ds(t * CH, CH)]],
                rows_v.at[rr],
                gsem.at[rr],
            )

        def wait_gather(rr):
            pltpu.make_async_copy(
                out_hbm.at[0, pl.ds(0, CH)], rows_v.at[rr], gsem.at[rr]
            ).wait()

        def fire_scatter(t, rr):
            r = row0 + t
            h = r // bt
            tb = r % bt
            pltpu.async_copy(
                rows_v.at[rr],
                out_hbm.at[h, pl.ds(tb * CH, CH)],
                ssem.at[rr],
            )

        def wait_scatter(rr):
            pltpu.make_async_copy(
                rows_v.at[rr], out_hbm.at[0, pl.ds(0, CH)], ssem.at[rr]
            ).wait()

        for t in range(R - G):
            fire_gather(t, t % R)
        for i in range(G):
            fire_gather(i + (R - G), (i + (R - G)) % R)
            wait_gather(i % R)
            fire_scatter(i, i % R)

        @pl.loop(0, steady // R)
        def _(o):
            t0 = G + o * R
            for k in range(R):
                t = t0 + k
                bpre = (G + k + (R - G)) % R
                wait_scatter(bpre)
                fire_gather(t + (R - G), bpre)
                b = (G + k) % R
                wait_gather(b)
                fire_scatter(t, b)

        for t in range(chunks_per_w - G, chunks_per_w):
            b = t % R
            wait_gather(b)
            fire_scatter(t, b)
        for rr in range(R):
            wait_scatter(rr)

    return gather_kernel


def kernel(x, weight):
    b, h = x.shape
    n, d = weight.shape
    hp = ((h + 7) // 8) * 8
    xp = jnp.pad(x, ((0, 0), (0, hp - h))) if hp != h else x
    xt = jnp.swapaxes(xp, 0, 1).astype(jnp.int32)
    idx = _make_detile(hp, b, d)(xt)
    wt = _make_detranspose(n, d)(jnp.swapaxes(weight, 0, 1))
    out = _make_gather(h, hp, b, d)(idx, wt.reshape(-1, d))
    return out.transpose(1, 0, 2)


# speedup vs baseline: 1.0851x; 1.0024x over previous
import functools

import jax
import jax.numpy as jnp
from jax import lax
from jax.experimental import pallas as pl
from jax.experimental.pallas import tpu as pltpu
from jax.experimental.pallas import tpu_sc as plsc

CH = 128
R = 8
G = 4


BLK = 2048


@functools.lru_cache(maxsize=None)
def _make_detile(hist_padded, batch, dim):
    ht = hist_padded // 8
    g = 128 // dim
    s = BLK // g
    sh = s.bit_length() - 1

    def body(x_ref, o_ref):
        r = x_ref[...]
        rp = (
            (r & ~(BLK - 1))
            + ((r & (s - 1)) << (g.bit_length() - 1))
            + ((r & (BLK - 1)) >> sh)
        )
        o_ref[...] = rp.reshape(8 * batch)

    return pl.pallas_call(
        body,
        grid=(ht,),
        in_specs=[pl.BlockSpec((8, batch), lambda i: (i, 0))],
        out_specs=pl.BlockSpec((8 * batch,), lambda i: (i,)),
        out_shape=jax.ShapeDtypeStruct((hist_padded * batch,), jnp.int32),
    )


@functools.lru_cache(maxsize=None)
def _make_detranspose(nemb, dim):
    g128 = 128 // dim
    s = BLK // g128
    grid = (nemb + BLK - 1) // BLK

    def body(w_ref, o_ref):
        t = jax.lax.dot_general(
            w_ref[...],
            jnp.eye(dim, dtype=jnp.float32),
            (((0,), (0,)), ((), ())),
            preferred_element_type=jnp.float32,
        )
        merged = jnp.concatenate(
            [t[k * s:(k + 1) * s] for k in range(g128)], axis=1
        )
        o_ref[...] = merged.reshape(BLK * dim)

    return pl.pallas_call(
        body,
        grid=(grid,),
        in_specs=[pl.BlockSpec((dim, BLK), lambda i: (0, i))],
        out_specs=pl.BlockSpec((BLK * dim,), lambda i: (i,)),
        out_shape=jax.ShapeDtypeStruct((grid * BLK * dim,), jnp.float32),
    )


@functools.lru_cache(maxsize=None)
def _make_gather(hist, hist_padded, batch, dim):
    mesh = plsc.VectorSubcoreMesh(core_axis_name="c", subcore_axis_name="s")
    nc, ns = mesh.num_cores, mesh.num_subcores
    nw = nc * ns
    bt = batch // CH
    num_chunks = hist * bt
    assert num_chunks % nw == 0
    chunks_per_w = num_chunks // nw
    steady = chunks_per_w - 2 * G
    assert steady % R == 0 and chunks_per_w > 2 * R

    @functools.partial(
        pl.kernel,
        out_type=jax.ShapeDtypeStruct((hist, batch, dim), jnp.float32),
        mesh=mesh,
        scratch_types=[
            pltpu.VMEM((chunks_per_w * CH,), jnp.int32),
            pltpu.VMEM((R, CH, dim), jnp.float32),
            pltpu.SemaphoreType.DMA((R,)),
            pltpu.SemaphoreType.DMA((R,)),
        ],
        compiler_params=pltpu.CompilerParams(use_tc_tiling_on_sc=False),
    )
    def gather_kernel(idx_hbm, table_hbm, out_hbm, idx_v, rows_v, gsem, ssem):
        wid = lax.axis_index("s") * nc + lax.axis_index("c")
        row0 = wid * chunks_per_w
        pltpu.sync_copy(
            idx_hbm.at[pl.ds(row0 * CH, chunks_per_w * CH)], idx_v
        )

        def fire_gather(t, rr):
            pltpu.async_copy(
                table_hbm.at[idx_v.at[pl.ds(t * CH, CH)]],
                rows_v.at[rr],
                gsem.at[rr],
            )

        def wait_gather(rr):
            pltpu.make_async_copy(
                out_hbm.at[0, pl.ds(0, CH)], rows_v.at[rr], gsem.at[rr]
            ).wait()

        def fire_scatter(t, rr):
            r = row0 + t
            h = r // bt
            tb = r % bt
            pltpu.async_copy(
                rows_v.at[rr],
                out_hbm.at[h, pl.ds(tb * CH, CH)],
                ssem.at[rr],
            )

        def wait_scatter(rr):
            pltpu.make_async_copy(
                rows_v.at[rr], out_hbm.at[0, pl.ds(0, CH)], ssem.at[rr]
            ).wait()

        for t in range(R - G):
            fire_gather(t, t % R)
        for i in range(G):
            fire_gather(i + (R - G), (i + (R - G)) % R)
            wait_gather(i % R)
            fire_scatter(i, i % R)

        @pl.loop(0, steady // R)
        def _(o):
            t0 = G + o * R
            for k in range(R):
                t = t0 + k
                bpre = (G + k + (R - G)) % R
                wait_scatter(bpre)
                fire_gather(t + (R - G), bpre)
                b = (G + k) % R
                wait_gather(b)
                fire_scatter(t, b)

        for t in range(chunks_per_w - G, chunks_per_w):
            b = t % R
            wait_gather(b)
            fire_scatter(t, b)
        for rr in range(R):
            wait_scatter(rr)

    return gather_kernel


def kernel(x, weight):
    b, h = x.shape
    n, d = weight.shape
    hp = ((h + 7) // 8) * 8
    xp = jnp.pad(x, ((0, 0), (0, hp - h))) if hp != h else x
    xt = jnp.swapaxes(xp, 0, 1).astype(jnp.int32)
    idx = _make_detile(hp, b, d)(xt)
    wt = _make_detranspose(n, d)(jnp.swapaxes(weight, 0, 1))
    out = _make_gather(h, hp, b, d)(idx, wt.reshape(-1, d))
    return out.transpose(1, 0, 2)
